# baseline (device time: 1622605 ns/iter reference)
import jax
import jax.numpy as jnp
from jax import lax
from jax.experimental import pallas as pl
from jax.experimental.pallas import tpu as pltpu

N_DEV = 32


def kernel(x, w_mat, scale_x, scale_w):
    m, k_per = x.shape
    n = w_mat.shape[1]
    m_per = m // N_DEV

    def body(x_ref, w_ref, sx_ref, sw_ref, out_ref,
             comm_ref, send_sems, recv_sems, credit_sem):
        my = lax.axis_index("i")
        left = lax.rem(my - 1 + N_DEV, N_DEV)
        right = lax.rem(my + 1, N_DEV)

        barrier_sem = pltpu.get_barrier_semaphore()
        for nbr in (left, right):
            pl.semaphore_signal(barrier_sem, inc=1, device_id=(nbr,),
                                device_id_type=pl.DeviceIdType.MESH)
        pl.semaphore_wait(barrier_sem, 2)

        def block(c):
            xb = x_ref[pl.ds(c * m_per, m_per), :]
            return jnp.dot(xb, w_ref[...], preferred_element_type=jnp.float32)

        comm_ref[0] = block(left)

        for h in range(N_DEV - 1):
            s_slot = h % 2
            r_slot = (h + 1) % 2
            if h >= 1:
                pl.semaphore_wait(credit_sem, 1)
            rdma = pltpu.make_async_remote_copy(
                src_ref=comm_ref.at[s_slot],
                dst_ref=comm_ref.at[r_slot],
                send_sem=send_sems.at[s_slot],
                recv_sem=recv_sems.at[r_slot],
                device_id=(right,),
                device_id_type=pl.DeviceIdType.MESH,
            )
            rdma.start()
            rdma.wait()
            if h <= N_DEV - 3:
                pl.semaphore_signal(credit_sem, inc=1, device_id=(left,),
                                    device_id_type=pl.DeviceIdType.MESH)
            c = lax.rem(my - 2 - h + 2 * N_DEV, N_DEV)
            if h < N_DEV - 2:
                comm_ref[r_slot] = comm_ref[r_slot] + block(c)
            else:
                scale = sx_ref[0] * sw_ref[0]
                out_ref[...] = (comm_ref[r_slot] + block(c)) * scale

    return pl.pallas_call(
        body,
        out_shape=jax.ShapeDtypeStruct((m_per, n), jnp.float32),
        in_specs=[
            pl.BlockSpec(memory_space=pltpu.VMEM),
            pl.BlockSpec(memory_space=pltpu.VMEM),
            pl.BlockSpec(memory_space=pltpu.SMEM),
            pl.BlockSpec(memory_space=pltpu.SMEM),
        ],
        out_specs=pl.BlockSpec(memory_space=pltpu.VMEM),
        scratch_shapes=[
            pltpu.VMEM((2, m_per, n), jnp.float32),
            pltpu.SemaphoreType.DMA((2,)),
            pltpu.SemaphoreType.DMA((2,)),
            pltpu.SemaphoreType.REGULAR,
        ],
        compiler_params=pltpu.CompilerParams(collective_id=0),
    )(x, w_mat, scale_x, scale_w)


# device time: 1478094 ns/iter; 1.0978x vs baseline; 1.0978x over previous
import jax
import jax.numpy as jnp
from jax import lax
from jax.experimental import pallas as pl
from jax.experimental.pallas import tpu as pltpu

N_DEV = 32


def kernel(x, w_mat, scale_x, scale_w):
    m, k_per = x.shape
    n = w_mat.shape[1]
    m_per = m // N_DEV
    n_half = n // 2

    def body(x_ref, w_ref, sx_ref, sw_ref, out_ref,
             commR_ref, commL_ref, sendR_sems, recvR_sems,
             sendL_sems, recvL_sems, creditR_sem, creditL_sem):
        my = lax.axis_index("i")
        left = lax.rem(my - 1 + N_DEV, N_DEV)
        right = lax.rem(my + 1, N_DEV)

        barrier_sem = pltpu.get_barrier_semaphore()
        for nbr in (left, right):
            pl.semaphore_signal(barrier_sem, inc=1, device_id=(nbr,),
                                device_id_type=pl.DeviceIdType.MESH)
        pl.semaphore_wait(barrier_sem, 2)

        def blockR(c):
            xb = x_ref[pl.ds(c * m_per, m_per), :]
            return jnp.dot(xb, w_ref[:, :n_half],
                           preferred_element_type=jnp.float32)

        def blockL(c):
            xb = x_ref[pl.ds(c * m_per, m_per), :]
            return jnp.dot(xb, w_ref[:, n_half:],
                           preferred_element_type=jnp.float32)

        commR_ref[0] = blockR(left)
        commL_ref[0] = blockL(right)

        for h in range(N_DEV - 1):
            s_slot = h % 2
            r_slot = (h + 1) % 2
            if h >= 1:
                pl.semaphore_wait(creditR_sem, 1)
                pl.semaphore_wait(creditL_sem, 1)
            rdmaR = pltpu.make_async_remote_copy(
                src_ref=commR_ref.at[s_slot],
                dst_ref=commR_ref.at[r_slot],
                send_sem=sendR_sems.at[s_slot],
                recv_sem=recvR_sems.at[r_slot],
                device_id=(right,),
                device_id_type=pl.DeviceIdType.MESH,
            )
            rdmaL = pltpu.make_async_remote_copy(
                src_ref=commL_ref.at[s_slot],
                dst_ref=commL_ref.at[r_slot],
                send_sem=sendL_sems.at[s_slot],
                recv_sem=recvL_sems.at[r_slot],
                device_id=(left,),
                device_id_type=pl.DeviceIdType.MESH,
            )
            rdmaR.start()
            rdmaL.start()
            rdmaR.wait()
            rdmaL.wait()
            if h <= N_DEV - 3:
                pl.semaphore_signal(creditR_sem, inc=1, device_id=(left,),
                                    device_id_type=pl.DeviceIdType.MESH)
                pl.semaphore_signal(creditL_sem, inc=1, device_id=(right,),
                                    device_id_type=pl.DeviceIdType.MESH)
            cR = lax.rem(my - 2 - h + 2 * N_DEV, N_DEV)
            cL = lax.rem(my + 2 + h, N_DEV)
            if h < N_DEV - 2:
                commR_ref[r_slot] = commR_ref[r_slot] + blockR(cR)
                commL_ref[r_slot] = commL_ref[r_slot] + blockL(cL)
            else:
                scale = sx_ref[0] * sw_ref[0]
                out_ref[:, :n_half] = (commR_ref[r_slot] + blockR(cR)) * scale
                out_ref[:, n_half:] = (commL_ref[r_slot] + blockL(cL)) * scale

    return pl.pallas_call(
        body,
        out_shape=jax.ShapeDtypeStruct((m_per, n), jnp.float32),
        in_specs=[
            pl.BlockSpec(memory_space=pltpu.VMEM),
            pl.BlockSpec(memory_space=pltpu.VMEM),
            pl.BlockSpec(memory_space=pltpu.SMEM),
            pl.BlockSpec(memory_space=pltpu.SMEM),
        ],
        out_specs=pl.BlockSpec(memory_space=pltpu.VMEM),
        scratch_shapes=[
            pltpu.VMEM((2, m_per, n_half), jnp.float32),
            pltpu.VMEM((2, m_per, n_half), jnp.float32),
            pltpu.SemaphoreType.DMA((2,)),
            pltpu.SemaphoreType.DMA((2,)),
            pltpu.SemaphoreType.DMA((2,)),
            pltpu.SemaphoreType.DMA((2,)),
            pltpu.SemaphoreType.REGULAR,
            pltpu.SemaphoreType.REGULAR,
        ],
        compiler_params=pltpu.CompilerParams(collective_id=0),
    )(x, w_mat, scale_x, scale_w)
